# Initial kernel scaffold; baseline (speedup 1.0000x reference)
import jax
import jax.numpy as jnp
from jax.experimental import pallas as pl


def kernel(x, edge_index, edge_attr, batch, action, params):
    raise NotImplementedError("write your pallas kernel here")


# speedup vs baseline: 1.8057x; 1.8057x over previous
import functools

import jax
import jax.numpy as jnp
from jax.experimental import pallas as pl

N_NODES = 10000
N_EDGES = 320000
D_FEAT = 128
D_EDGE = 16
HIDDEN = 16
NUM_GRAPHS = 16
ACTION_DIM = 8
NUM_LAYERS = 4


def _proj_body(x_ref, w_ref, b_ref, o_ref):
    o_ref[...] = jnp.dot(x_ref[...], w_ref[...],
                         preferred_element_type=jnp.float32) + b_ref[...]


def _proj(x, w, b, block_rows):
    n = x.shape[0]
    grid = n // block_rows
    return pl.pallas_call(
        _proj_body,
        grid=(grid,),
        in_specs=[
            pl.BlockSpec((block_rows, x.shape[1]), lambda i: (i, 0)),
            pl.BlockSpec((w.shape[0], w.shape[1]), lambda i: (0, 0)),
            pl.BlockSpec((w.shape[1],), lambda i: (0,)),
        ],
        out_specs=pl.BlockSpec((block_rows, w.shape[1]), lambda i: (i, 0)),
        out_shape=jax.ShapeDtypeStruct((n, w.shape[1]), jnp.float32),
    )(x, w, b)


def _layer_norm(h, g, b, eps=1e-5):
    mu = jnp.mean(h, axis=-1, keepdims=True)
    var = jnp.var(h, axis=-1, keepdims=True)
    return (h - mu) / jnp.sqrt(var + eps) * g + b


def _gen_conv(h, src, dst, eattr, cp, num_nodes):
    msg = h[src] + eattr
    msg = jax.nn.relu(msg) + 1e-7
    p = jnp.exp(cp['t'] * msg)
    denom = jax.ops.segment_sum(p, dst, num_segments=num_nodes)
    wsum = jax.ops.segment_sum(msg * p, dst, num_segments=num_nodes)
    agg = wsum / (denom + 1e-16)
    out = agg + h
    z = out @ cp['w1'] + cp['b1']
    z = _layer_norm(z, cp['ln_g'], cp['ln_b'])
    z = jax.nn.relu(z)
    z = z @ cp['w2'] + cp['b2']
    return z


def kernel(x, edge_index, edge_attr, batch, action, params):
    h = _proj(x, params['node_w'], params['node_b'], 1000)
    e = _proj(edge_attr, params['edge_w'], params['edge_b'], 1000)
    src = edge_index[0]
    dst = edge_index[1]
    n = h.shape[0]
    h = _gen_conv(h, src, dst, e, params['convs'][0], n)
    for i in range(1, NUM_LAYERS):
        r = _layer_norm(h, params['norms'][i]['g'], params['norms'][i]['b'])
        r = jax.nn.relu(r)
        r = _gen_conv(r, src, dst, e, params['convs'][i], n)
        h = h + r
    h = _layer_norm(h, params['norms'][0]['g'], params['norms'][0]['b'])
    h = jax.nn.relu(h)
    gmax = jax.ops.segment_max(h, batch, num_segments=NUM_GRAPHS)
    gmax = jnp.where(jnp.isfinite(gmax), gmax, 0.0)
    gsum = jax.ops.segment_sum(h, batch, num_segments=NUM_GRAPHS)
    cnt = jax.ops.segment_sum(jnp.ones((n,), jnp.float32), batch,
                              num_segments=NUM_GRAPHS)
    gmean = gsum / jnp.maximum(cnt, 1.0)[:, None]
    mol = jnp.concatenate([gmax, gmean], axis=1)
    fp = jax.nn.relu(mol @ params['pin_w'] + params['pin_b'])
    pol = jnp.concatenate([fp, action], axis=1) @ params['ph_w'] + params['ph_b']
    pol = jax.nn.relu(pol) @ params['pout_w'] + params['pout_b']
    return pol


# speedup vs baseline: 11.3362x; 6.2781x over previous
import functools

import jax
import jax.numpy as jnp
from jax import lax
from jax.experimental import pallas as pl
from jax.experimental.pallas import tpu as pltpu
from jax.experimental.pallas import tpu_sc as plsc

N_NODES = 10000
N_EDGES = 320000
D_FEAT = 128
D_EDGE = 16
HIDDEN = 16
NUM_GRAPHS = 16
ACTION_DIM = 8
NUM_LAYERS = 4

NUM_TILES = 32
CH = 128
NCHUNK = N_EDGES // CH
BASE_CHUNKS = NCHUNK // NUM_TILES
EXTRA_TILES = NCHUNK - BASE_CHUNKS * NUM_TILES
N_PAD = 10112
RPT = N_PAD // 16


def _mp_body(hin, srcr, dstr, er, tarr, pt_out0, pt_out1, wt_out0, wt_out1,
             pt_s, wt_s, sidx, didx, hrows, erows, prows, wrows, obuf,
             tbuf, sem):
    c = lax.axis_index("c")
    s = lax.axis_index("s")
    wid = c * 16 + s

    pltpu.sync_copy(tarr, tbuf)
    tv = tbuf[...]

    zero16 = jnp.zeros((16,), jnp.float32)

    def _zrow(j, carry):
        obuf[j, :] = zero16
        return carry

    lax.fori_loop(0, RPT, _zrow, 0)
    pltpu.sync_copy(obuf, pt_s.at[pl.ds(s * RPT, RPT), :])
    pltpu.sync_copy(obuf, wt_s.at[pl.ds(s * RPT, RPT), :])
    plsc.subcore_barrier()

    nch = BASE_CHUNKS + jnp.where(wid < EXTRA_TILES, 1, 0)

    def _chunk(i, carry):
        base = (wid + i * NUM_TILES) * CH
        pltpu.sync_copy(srcr.at[pl.ds(base, CH)], sidx)
        pltpu.sync_copy(dstr.at[pl.ds(base, CH)], didx)
        pltpu.sync_copy(er.at[pl.ds(base, CH), :], erows)
        pltpu.async_copy(hin.at[sidx], hrows, sem).wait()

        def _row(j, rc):
            m = jnp.maximum(hrows[j, :] + erows[j, :], 0.0) + 1e-7
            p = jnp.exp(tv * m)
            prows[j, :] = p
            wrows[j, :] = m * p
            return rc

        lax.fori_loop(0, CH, _row, 0)
        pltpu.sync_copy(prows, pt_s.at[didx], add=True)
        pltpu.sync_copy(wrows, wt_s.at[didx], add=True)
        return carry

    lax.fori_loop(0, nch, _chunk, 0)
    plsc.subcore_barrier()

    @pl.when(c == 0)
    def _out0():
        pltpu.sync_copy(pt_s.at[pl.ds(s * RPT, RPT), :], obuf)
        pltpu.sync_copy(obuf, pt_out0.at[pl.ds(s * RPT, RPT), :])
        pltpu.sync_copy(wt_s.at[pl.ds(s * RPT, RPT), :], obuf)
        pltpu.sync_copy(obuf, wt_out0.at[pl.ds(s * RPT, RPT), :])

    @pl.when(c == 1)
    def _out1():
        pltpu.sync_copy(pt_s.at[pl.ds(s * RPT, RPT), :], obuf)
        pltpu.sync_copy(obuf, pt_out1.at[pl.ds(s * RPT, RPT), :])
        pltpu.sync_copy(wt_s.at[pl.ds(s * RPT, RPT), :], obuf)
        pltpu.sync_copy(obuf, wt_out1.at[pl.ds(s * RPT, RPT), :])


_sc_mesh = plsc.VectorSubcoreMesh(core_axis_name="c", subcore_axis_name="s")

_mp_call = pl.kernel(
    _mp_body,
    out_type=[
        jax.ShapeDtypeStruct((N_PAD, HIDDEN), jnp.float32),
        jax.ShapeDtypeStruct((N_PAD, HIDDEN), jnp.float32),
        jax.ShapeDtypeStruct((N_PAD, HIDDEN), jnp.float32),
        jax.ShapeDtypeStruct((N_PAD, HIDDEN), jnp.float32),
    ],
    mesh=_sc_mesh,
    scratch_types=[
        pltpu.VMEM_SHARED((N_PAD, HIDDEN), jnp.float32),
        pltpu.VMEM_SHARED((N_PAD, HIDDEN), jnp.float32),
        pltpu.VMEM((CH,), jnp.int32),
        pltpu.VMEM((CH,), jnp.int32),
        pltpu.VMEM((CH, HIDDEN), jnp.float32),
        pltpu.VMEM((CH, HIDDEN), jnp.float32),
        pltpu.VMEM((CH, HIDDEN), jnp.float32),
        pltpu.VMEM((CH, HIDDEN), jnp.float32),
        pltpu.VMEM((RPT, HIDDEN), jnp.float32),
        pltpu.VMEM((16,), jnp.float32),
        pltpu.SemaphoreType.DMA,
    ],
    compiler_params=pltpu.CompilerParams(use_tc_tiling_on_sc=False),
)


def _proj_body(x_ref, w_ref, b_ref, o_ref):
    o_ref[...] = jnp.dot(x_ref[...], w_ref[...],
                         preferred_element_type=jnp.float32) + b_ref[...]


def _proj(x, w, b, block_rows):
    n = x.shape[0]
    return pl.pallas_call(
        _proj_body,
        grid=(n // block_rows,),
        in_specs=[
            pl.BlockSpec((block_rows, x.shape[1]), lambda i: (i, 0)),
            pl.BlockSpec((w.shape[0], w.shape[1]), lambda i: (0, 0)),
            pl.BlockSpec((w.shape[1],), lambda i: (0,)),
        ],
        out_specs=pl.BlockSpec((block_rows, w.shape[1]), lambda i: (i, 0)),
        out_shape=jax.ShapeDtypeStruct((n, w.shape[1]), jnp.float32),
    )(x, w, b)


def _ln(h, g, b, eps=1e-5):
    mu = jnp.mean(h, axis=-1, keepdims=True)
    var = jnp.mean((h - mu) ** 2, axis=-1, keepdims=True)
    return (h - mu) / jnp.sqrt(var + eps) * g + b


def _layer_body(pt0, pt1, wt0, wt1, hin, hres, w1, b1, lng, lnb, w2, b2,
                ng, nb, hnew_ref, rnext_ref):
    den = pt0[...] + pt1[...]
    num = wt0[...] + wt1[...]
    agg = num / (den + 1e-16)
    out = agg + hin[...]
    z = jnp.dot(out, w1[...], preferred_element_type=jnp.float32) + b1[...]
    z = _ln(z, lng[...], lnb[...])
    z = jnp.maximum(z, 0.0)
    z = jnp.dot(z, w2[...], preferred_element_type=jnp.float32) + b2[...]
    hnew = hres[...] + z
    hnew_ref[...] = hnew
    rnext_ref[...] = jnp.maximum(_ln(hnew, ng[...], nb[...]), 0.0)


def _layer_call(pt0, pt1, wt0, wt1, hin, hres, cp, ng, nb, block_rows=1000):
    n = N_NODES
    grid = n // block_rows
    rows = lambda i: (i, 0)
    full2 = lambda shape: pl.BlockSpec(shape, lambda i: (0, 0))
    full1 = lambda shape: pl.BlockSpec(shape, lambda i: (0,))
    return pl.pallas_call(
        _layer_body,
        grid=(grid,),
        in_specs=[
            pl.BlockSpec((block_rows, HIDDEN), rows),
            pl.BlockSpec((block_rows, HIDDEN), rows),
            pl.BlockSpec((block_rows, HIDDEN), rows),
            pl.BlockSpec((block_rows, HIDDEN), rows),
            pl.BlockSpec((block_rows, HIDDEN), rows),
            pl.BlockSpec((block_rows, HIDDEN), rows),
            full2((HIDDEN, 2 * HIDDEN)),
            full1((2 * HIDDEN,)),
            full1((2 * HIDDEN,)),
            full1((2 * HIDDEN,)),
            full2((2 * HIDDEN, HIDDEN)),
            full1((HIDDEN,)),
            full1((HIDDEN,)),
            full1((HIDDEN,)),
        ],
        out_specs=[
            pl.BlockSpec((block_rows, HIDDEN), rows),
            pl.BlockSpec((block_rows, HIDDEN), rows),
        ],
        out_shape=[
            jax.ShapeDtypeStruct((n, HIDDEN), jnp.float32),
            jax.ShapeDtypeStruct((n, HIDDEN), jnp.float32),
        ],
    )(pt0, pt1, wt0, wt1, hin, hres, cp['w1'], cp['b1'], cp['ln_g'],
      cp['ln_b'], cp['w2'], cp['b2'], ng, nb)


def _pool_body(h_ref, b_ref, action, pw_top, pw_bot, pin_b, phw_a, phw_b,
               ph_b, pout_w, pout_b, out_ref, gmax_acc, gsum_acc, cnt_acc):
    i = pl.program_id(0)

    @pl.when(i == 0)
    def _init():
        gmax_acc[...] = jnp.full((NUM_GRAPHS, HIDDEN), -jnp.inf, jnp.float32)
        gsum_acc[...] = jnp.zeros((NUM_GRAPHS, HIDDEN), jnp.float32)
        cnt_acc[...] = jnp.zeros((NUM_GRAPHS, HIDDEN), jnp.float32)

    h = h_ref[...]
    bids = b_ref[0, 0, :]
    onehot = (bids[:, None] ==
              lax.broadcasted_iota(jnp.int32, (1, NUM_GRAPHS), 1)
              ).astype(jnp.float32)
    gsum_acc[...] += lax.dot_general(
        onehot, h, (((0,), (0,)), ((), ())),
        preferred_element_type=jnp.float32)
    cnt_acc[...] += lax.dot_general(
        onehot, jnp.ones_like(h), (((0,), (0,)), ((), ())),
        preferred_element_type=jnp.float32)
    mask = onehot > 0.5
    for g in range(NUM_GRAPHS):
        hm = jnp.where(mask[:, g:g + 1], h, -jnp.inf)
        gmax_acc[g:g + 1, :] = jnp.maximum(
            gmax_acc[g:g + 1, :], jnp.max(hm, axis=0, keepdims=True))

    gmax = gmax_acc[...]
    gmax = jnp.where(jnp.isfinite(gmax), gmax, 0.0)
    gmean = gsum_acc[...] / jnp.maximum(cnt_acc[...], 1.0)
    fp = jnp.dot(gmax, pw_top[...], preferred_element_type=jnp.float32)
    fp += jnp.dot(gmean, pw_bot[...], preferred_element_type=jnp.float32)
    fp = jnp.maximum(fp + pin_b[...], 0.0)
    t = jnp.dot(fp, phw_a[...], preferred_element_type=jnp.float32)
    t += jnp.dot(action[...], phw_b[...], preferred_element_type=jnp.float32)
    t = jnp.maximum(t + ph_b[...], 0.0)
    out_ref[...] = (jnp.dot(t, pout_w[...], preferred_element_type=jnp.float32)
                    + pout_b[...])


def _pool_call(h, batch3, action, params, block_rows=1000):
    grid = N_NODES // block_rows
    pin_w, pin_b = params['pin_w'], params['pin_b']
    ph_w, ph_b = params['ph_w'], params['ph_b']
    pout_w, pout_b = params['pout_w'], params['pout_b']
    full2 = lambda shape: pl.BlockSpec(shape, lambda i: (0, 0))
    full1 = lambda shape: pl.BlockSpec(shape, lambda i: (0,))
    return pl.pallas_call(
        _pool_body,
        grid=(grid,),
        in_specs=[
            pl.BlockSpec((block_rows, HIDDEN), lambda i: (i, 0)),
            pl.BlockSpec((1, 1, block_rows), lambda i: (i, 0, 0)),
            full2((NUM_GRAPHS, ACTION_DIM)),
            full2((HIDDEN, 128)),
            full2((HIDDEN, 128)),
            full1((128,)),
            full2((128, 10)),
            full2((ACTION_DIM, 10)),
            full1((10,)),
            full2((10, 1)),
            full1((1,)),
        ],
        out_specs=pl.BlockSpec((NUM_GRAPHS, 1), lambda i: (0, 0)),
        out_shape=jax.ShapeDtypeStruct((NUM_GRAPHS, 1), jnp.float32),
        scratch_shapes=[
            pltpu.VMEM((NUM_GRAPHS, HIDDEN), jnp.float32),
            pltpu.VMEM((NUM_GRAPHS, HIDDEN), jnp.float32),
            pltpu.VMEM((NUM_GRAPHS, HIDDEN), jnp.float32),
        ],
    )(h, batch3, action, pin_w[:HIDDEN], pin_w[HIDDEN:], pin_b,
      ph_w[:128], ph_w[128:], ph_b, pout_w, pout_b)


def kernel(x, edge_index, edge_attr, batch, action, params):
    h = _proj(x, params['node_w'], params['node_b'], 1000)
    e = _proj(edge_attr, params['edge_w'], params['edge_b'], 4000)
    src = edge_index[0]
    dst = edge_index[1]
    zeros = jnp.zeros((N_NODES, HIDDEN), jnp.float32)
    batch3 = batch.reshape(10, 1, N_NODES // 10)

    hin = h
    hres = zeros
    for i in range(NUM_LAYERS):
        cp = params['convs'][i]
        tarr = jnp.full((16,), cp['t'], jnp.float32)
        pt0, pt1, wt0, wt1 = _mp_call(hin, src, dst, e, tarr)
        nrm = params['norms'][(i + 1) % NUM_LAYERS]
        hnew, rnext = _layer_call(pt0, pt1, wt0, wt1, hin, hres, cp,
                                  nrm['g'], nrm['b'])
        hin = rnext
        hres = hnew

    return _pool_call(hin, batch3, action, params)
